# baseline (speedup 1.0000x reference)
import functools

import jax
import jax.numpy as jnp
from jax.experimental import pallas as pl
from jax.experimental.pallas import tpu as pltpu

H = 128
DEPTH = 4


def _init_mm_kernel(fb_ref, wi_ref, out_ref):
    out_ref[...] = jnp.dot(fb_ref[...], wi_ref[...],
                           preferred_element_type=jnp.float32)


def _layer_mm_kernel(fb_ref, t_ref, wi_ref, wh_ref, out_ref):
    acc = jnp.dot(fb_ref[...], wi_ref[...], preferred_element_type=jnp.float32)
    acc = acc + jnp.dot(t_ref[...], wh_ref[...],
                        preferred_element_type=jnp.float32)
    out_ref[...] = jnp.maximum(acc, 0.0)


def _final_mm_kernel(fa_ref, am_ref, wo1_ref, wo2_ref, mask_ref, out_ref):
    acc = jnp.dot(fa_ref[...], wo1_ref[...], preferred_element_type=jnp.float32)
    acc = acc + jnp.dot(am_ref[...], wo2_ref[...],
                        preferred_element_type=jnp.float32)
    out_ref[...] = jnp.maximum(acc, 0.0) * mask_ref[...]


def _init_mm(f_bonds, W_i, br):
    nb = f_bonds.shape[0]
    k = f_bonds.shape[1]
    return pl.pallas_call(
        _init_mm_kernel,
        grid=(nb // br,),
        in_specs=[
            pl.BlockSpec((br, k), lambda i: (i, 0)),
            pl.BlockSpec((k, H), lambda i: (0, 0)),
        ],
        out_specs=pl.BlockSpec((br, H), lambda i: (i, 0)),
        out_shape=jax.ShapeDtypeStruct((nb, H), jnp.float32),
    )(f_bonds, W_i)


def _layer_mm(f_bonds, t, W_i, W_h, br):
    nb = f_bonds.shape[0]
    k = f_bonds.shape[1]
    return pl.pallas_call(
        _layer_mm_kernel,
        grid=(nb // br,),
        in_specs=[
            pl.BlockSpec((br, k), lambda i: (i, 0)),
            pl.BlockSpec((br, H), lambda i: (i, 0)),
            pl.BlockSpec((k, H), lambda i: (0, 0)),
            pl.BlockSpec((H, H), lambda i: (0, 0)),
        ],
        out_specs=pl.BlockSpec((br, H), lambda i: (i, 0)),
        out_shape=jax.ShapeDtypeStruct((nb, H), jnp.float32),
    )(f_bonds, t, W_i, W_h)


def _final_mm(f_atoms, a_msg, W_o, mask, br):
    na = f_atoms.shape[0]
    fd = f_atoms.shape[1]
    return pl.pallas_call(
        _final_mm_kernel,
        grid=(na // br,),
        in_specs=[
            pl.BlockSpec((br, fd), lambda i: (i, 0)),
            pl.BlockSpec((br, H), lambda i: (i, 0)),
            pl.BlockSpec((fd, H), lambda i: (0, 0)),
            pl.BlockSpec((H, H), lambda i: (0, 0)),
            pl.BlockSpec((br, 1), lambda i: (i, 0)),
        ],
        out_specs=pl.BlockSpec((br, H), lambda i: (i, 0)),
        out_shape=jax.ShapeDtypeStruct((na, H), jnp.float32),
    )(f_atoms, a_msg, W_o[:fd], W_o[fd:], mask)


def kernel(f_atoms, f_bonds, a2b, b2a, b2revb, mask, W_i, W_h, W_o):
    inp = _init_mm(f_bonds, W_i, br=3200)
    message = jnp.maximum(inp, 0.0)
    for _ in range(DEPTH - 1):
        a_msg = jnp.take(message, a2b, axis=0).sum(axis=1)
        t = jnp.take(a_msg, b2a, axis=0) - jnp.take(message, b2revb, axis=0)
        message = _layer_mm(f_bonds, t, W_i, W_h, br=3200)
    a_msg = jnp.take(message, a2b, axis=0).sum(axis=1)
    return _final_mm(f_atoms, a_msg, W_o, mask, br=2000)
